# baseline (device time: 178787 ns/iter reference)
import numpy as np

import jax
import jax.numpy as jnp
from jax import lax
from jax.experimental import pallas as pl
from jax.experimental.pallas import tpu as pltpu

N_DEV = 4
B, SQ, SKV_SH, DH = 2, 512, 512, 64
H_SH = 8
HID = H_SH * DH
SKV = N_DEV * SKV_SH
D_OUT = 768
BLK = 64
PAIR = 2 * DH

BF = jnp.bfloat16
F32 = jnp.float32


def _global_mask() -> np.ndarray:
    qb = (np.arange(SQ) // BLK)[:, None]
    kb = (np.arange(SKV) // BLK)[None, :]
    m = (qb == kb) | (kb == 0) | ((qb + kb) % 3 == 0)
    return m.astype(np.float32)


def kernel(x, Wq, K_ext, V_ext, Wo):
    K2 = K_ext.reshape(B, SKV_SH, N_DEV * HID).astype(BF)
    V2 = V_ext.reshape(B, SKV_SH, N_DEV * HID).astype(BF)
    x16 = x.astype(BF)
    wq16 = Wq.astype(BF)
    wo16 = Wo.astype(BF)
    mask = jnp.asarray(_global_mask(), dtype=BF)

    def body(x_ref, wq_ref, k_ref, v_ref, wo_ref, mask_ref, out_ref,
             kvall, qbuf, mbuf, lbuf, cbuf, osend, orecv,
             kv_send_sems, kv_recv_sems, o_send_sems, o_recv_sems):
        my = lax.axis_index("i")

        kv_rdmas = {}
        for d in range(1, N_DEV):
            dst = lax.rem(my + d, N_DEV)
            rk = pltpu.make_async_remote_copy(
                src_ref=k_ref.at[:, :, pl.ds(dst * HID, HID)],
                dst_ref=kvall.at[d - 1, 0],
                send_sem=kv_send_sems.at[0, d - 1],
                recv_sem=kv_recv_sems.at[0, d - 1],
                device_id=(dst,),
                device_id_type=pl.DeviceIdType.MESH,
            )
            rv = pltpu.make_async_remote_copy(
                src_ref=v_ref.at[:, :, pl.ds(dst * HID, HID)],
                dst_ref=kvall.at[d - 1, 1],
                send_sem=kv_send_sems.at[1, d - 1],
                recv_sem=kv_recv_sems.at[1, d - 1],
                device_id=(dst,),
                device_id_type=pl.DeviceIdType.MESH,
            )
            rk.start()
            rv.start()
            kv_rdmas[d] = (rk, rv)

        wo_bf = wo_ref[:, :]
        for b in range(B):
            qbuf[b] = lax.dot_general(
                x_ref[b], wq_ref[:, :], (((1,), (0,)), ((), ())),
                preferred_element_type=F32).astype(BF)

        def run_phase(d, is_first, is_last):
            src = lax.rem(my - d + N_DEV, N_DEV)
            for b in range(B):
                def pair_step(hp, carry, b=b, d=d,
                              is_first=is_first, is_last=is_last, src=src):
                    hs = hp * PAIR
                    q2 = qbuf[b, :, pl.ds(hs, PAIR)]
                    if d == 0:
                        kc2 = k_ref[b, :, pl.ds(my * HID + hs, PAIR)]
                        vc2 = v_ref[b, :, pl.ds(my * HID + hs, PAIR)]
                    else:
                        kc2 = kvall[d - 1, 0, b, :, pl.ds(hs, PAIR)]
                        vc2 = kvall[d - 1, 1, b, :, pl.ds(hs, PAIR)]
                    mk = mask_ref[:, pl.ds(src * SKV_SH, SKV_SH)]
                    if not is_first:
                        ml2 = mbuf[b, :, pl.ds(hs, PAIR)]
                        ll2 = lbuf[b, :, pl.ds(hs, PAIR)]
                        cc2 = cbuf[b, :, pl.ds(hs, PAIR)]
                    cs, ms, ls = [], [], []
                    for sub in range(2):
                        lo, hi = sub * DH, (sub + 1) * DH
                        q = q2[:, lo:hi]
                        kc = kc2[:, lo:hi]
                        vc = vc2[:, lo:hi]
                        s = lax.dot_general(q, kc,
                                            (((1,), (1,)), ((), ())),
                                            preferred_element_type=F32) * 0.125
                        s = jnp.where(mk > 0.5, s, -1e9)
                        smax = s.max(axis=1, keepdims=True)
                        if is_first:
                            m_new = smax
                            p = jnp.exp(s - m_new)
                            l = p.sum(axis=1, keepdims=True)
                            ctx = lax.dot_general(
                                p.astype(BF), vc, (((1,), (0,)), ((), ())),
                                preferred_element_type=F32)
                        else:
                            m_old = ml2[:, lo:lo + 1]
                            m_new = jnp.maximum(m_old, smax)
                            alpha = jnp.exp(m_old - m_new)
                            p = jnp.exp(s - m_new)
                            l = ll2[:, lo:lo + 1] * alpha + p.sum(
                                axis=1, keepdims=True)
                            ctx = cc2[:, lo:hi] * alpha + lax.dot_general(
                                p.astype(BF), vc, (((1,), (0,)), ((), ())),
                                preferred_element_type=F32)
                        if is_last:
                            ctx = ctx / l
                        cs.append(ctx)
                        ms.append(jnp.broadcast_to(m_new, (SQ, DH)))
                        ls.append(jnp.broadcast_to(l, (SQ, DH)))
                    cbuf[b, :, pl.ds(hs, PAIR)] = jnp.concatenate(cs, axis=1)
                    if not is_last:
                        mbuf[b, :, pl.ds(hs, PAIR)] = jnp.concatenate(
                            ms, axis=1)
                        lbuf[b, :, pl.ds(hs, PAIR)] = jnp.concatenate(
                            ls, axis=1)
                    return carry

                lax.fori_loop(0, H_SH // 2, pair_step, 0)

        run_phase(0, True, False)
        for d, is_last in ((1, False), (3, False), (2, True)):
            rk, rv = kv_rdmas[d]
            rk.wait_recv()
            rv.wait_recv()
            run_phase(d, False, is_last)

        o_rdmas = []
        for b in range(B):
            out_ref[b] = lax.dot_general(
                cbuf[b].astype(BF), wo_bf, (((1,), (0,)), ((), ())),
                preferred_element_type=F32)
            osend[b] = out_ref[b].astype(BF)
            for d in range(1, N_DEV):
                dst = lax.rem(my + d, N_DEV)
                r = pltpu.make_async_remote_copy(
                    src_ref=osend.at[b],
                    dst_ref=orecv.at[d - 1, b],
                    send_sem=o_send_sems.at[d - 1, b],
                    recv_sem=o_recv_sems.at[d - 1, b],
                    device_id=(dst,),
                    device_id_type=pl.DeviceIdType.MESH,
                )
                r.start()
                o_rdmas.append(r)

        for rk, rv in kv_rdmas.values():
            rk.wait_send()
            rv.wait_send()
        for r in o_rdmas:
            r.wait_send()
            r.wait_recv()
        out_ref[:, :, :] = (out_ref[:, :, :]
                            + orecv[0].astype(F32)
                            + orecv[1].astype(F32)
                            + orecv[2].astype(F32))

    return pl.pallas_call(
        body,
        out_shape=jax.ShapeDtypeStruct((B, SQ, D_OUT), F32),
        in_specs=[pl.BlockSpec(memory_space=pltpu.VMEM)] * 6,
        out_specs=pl.BlockSpec(memory_space=pltpu.VMEM),
        scratch_shapes=[
            pltpu.VMEM((N_DEV - 1, 2, B, SKV_SH, HID), BF),
            pltpu.VMEM((B, SQ, HID), BF),
            pltpu.VMEM((B, SQ, HID), F32),
            pltpu.VMEM((B, SQ, HID), F32),
            pltpu.VMEM((B, SQ, HID), F32),
            pltpu.VMEM((B, SQ, D_OUT), BF),
            pltpu.VMEM((N_DEV - 1, B, SQ, D_OUT), BF),
            pltpu.SemaphoreType.DMA((2, N_DEV - 1)),
            pltpu.SemaphoreType.DMA((2, N_DEV - 1)),
            pltpu.SemaphoreType.DMA((N_DEV - 1, B)),
            pltpu.SemaphoreType.DMA((N_DEV - 1, B)),
        ],
        compiler_params=pltpu.CompilerParams(
            vmem_limit_bytes=100 * 1024 * 1024,
        ),
    )(x16, wq16, K2, V2, wo16, mask)
